# Initial kernel scaffold; baseline (speedup 1.0000x reference)
import functools

import jax
import jax.numpy as jnp
from jax import lax
from jax.experimental import pallas as pl
from jax.experimental.pallas import tpu as pltpu
from jax.experimental.pallas import tpu_sc as plsc

_NE = 8192
_D = 256
_TM = 256


def _argmin_body(x_ref, e_ref, out_ref, nrm_ref):
    t = pl.program_id(0)

    @pl.when(t == 0)
    def _():
        e = e_ref[...]
        nrm_ref[...] = jnp.sum(e * e, axis=1, keepdims=True)

    dot = lax.dot_general(
        e_ref[...], x_ref[...], (((1,), (1,)), ((), ())),
        preferred_element_type=jnp.float32)
    scores = nrm_ref[...] - 2.0 * dot
    bmin = jnp.min(scores, axis=0, keepdims=True)
    iota = lax.broadcasted_iota(jnp.int32, scores.shape, 0)
    cand = jnp.where(scores == bmin, iota, _NE)
    out_ref[0] = jnp.min(cand, axis=0, keepdims=True)


def _tc_argmin(flat, embeddings, *, interpret=False):
    n_tokens = flat.shape[0]
    n_blocks = n_tokens // _TM
    out = pl.pallas_call(
        _argmin_body,
        grid=(n_blocks,),
        in_specs=[
            pl.BlockSpec((_TM, _D), lambda t: (t, 0)),
            pl.BlockSpec((_NE, _D), lambda t: (0, 0)),
        ],
        out_specs=pl.BlockSpec((1, 1, _TM), lambda t: (t, 0, 0)),
        out_shape=jax.ShapeDtypeStruct((n_blocks, 1, _TM), jnp.int32),
        scratch_shapes=[pltpu.VMEM((_NE, 1), jnp.float32)],
        compiler_params=pltpu.CompilerParams(
            dimension_semantics=("arbitrary",)),
        interpret=interpret,
    )(flat, embeddings)
    return out.reshape(-1)


_CH = 128


def _sc_gather(table, idx):
    info = plsc.get_sparse_core_info()
    nw = info.num_cores * info.num_subcores
    b = idx.shape[0]
    d = table.shape[1]
    bpw = b // nw
    mesh = plsc.VectorSubcoreMesh(core_axis_name="c", subcore_axis_name="s")

    @functools.partial(
        pl.kernel, mesh=mesh,
        out_type=jax.ShapeDtypeStruct((b, d), jnp.float32),
        scratch_types=[
            pltpu.VMEM((_CH,), jnp.int32),
            pltpu.VMEM((_CH, d), jnp.float32),
            pltpu.SemaphoreType.DMA,
        ],
    )
    def gk(table_hbm, idx_hbm, out_hbm, idx_v, rows_v, sem):
        wid = lax.axis_index("s") * info.num_cores + lax.axis_index("c")
        base = wid * bpw

        def body(i, carry):
            off = pl.multiple_of(base + i * _CH, _CH)
            pltpu.sync_copy(idx_hbm.at[pl.ds(off, _CH)], idx_v)
            pltpu.async_copy(table_hbm.at[idx_v], rows_v, sem).wait()
            pltpu.sync_copy(rows_v, out_hbm.at[pl.ds(off, _CH)])
            return carry

        lax.fori_loop(0, bpw // _CH, body, 0)

    return gk(table, idx)


def kernel(inputs, embeddings):
    lat = jnp.transpose(inputs, (0, 2, 3, 1))
    flat = lat.reshape(-1, _D)
    idx = _tc_argmin(flat, embeddings)
    qflat = _sc_gather(embeddings, idx)
    qlat = qflat.reshape(lat.shape)
    quant = lat + lax.stop_gradient(qlat - lat)
    return (jnp.transpose(quant, (0, 3, 1, 2)), lat, qlat)


# speedup vs baseline: 6.7750x; 6.7750x over previous
import functools

import jax
import jax.numpy as jnp
from jax import lax
from jax.experimental import pallas as pl
from jax.experimental.pallas import tpu as pltpu
from jax.experimental.pallas import tpu_sc as plsc

_NE = 8192
_D = 256
_TM = 256


_BOUNDS = ((0, 2736), (2736, 5472), (5472, 8192))


def _argmin_body(xb_ref, eb_ref, xnrm_ref, enrm_ref, out_ref):
    dot = lax.dot_general(
        xb_ref[...], eb_ref[...], (((1,), (1,)), ((), ())),
        preferred_element_type=jnp.float32)
    scores = (xnrm_ref[...] + enrm_ref[...]) - 2.0 * dot
    cols = lax.broadcasted_iota(jnp.int32, scores.shape, 1)
    run_v = jnp.full((_TM, 1), jnp.inf, jnp.float32)
    run_i = jnp.zeros((_TM, 1), jnp.int32)
    for lo, hi in _BOUNDS:
        inside = (cols >= lo) & (cols < hi)
        sub = jnp.where(inside, scores, jnp.inf)
        bmin = jnp.min(sub, axis=1, keepdims=True)
        barg = jnp.min(jnp.where(sub == bmin, cols, _NE), axis=1,
                       keepdims=True)
        take = (bmin < run_v) | ((bmin == run_v) & (barg < run_i))
        run_i = jnp.where(take, barg, run_i)
        run_v = jnp.where(take, bmin, run_v).astype(jnp.bfloat16).astype(
            jnp.float32)
    out_ref[...] = run_i


def _tc_argmin(flat_bf, emb_bf, xnrm, enrm, *, interpret=False):
    n_tokens = flat_bf.shape[0]
    n_blocks = n_tokens // _TM
    out = pl.pallas_call(
        _argmin_body,
        grid=(n_blocks,),
        in_specs=[
            pl.BlockSpec((_TM, _D), lambda t: (t, 0)),
            pl.BlockSpec((_NE, _D), lambda t: (0, 0)),
            pl.BlockSpec((_TM, 1), lambda t: (t, 0)),
            pl.BlockSpec((1, _NE), lambda t: (0, 0)),
        ],
        out_specs=pl.BlockSpec((_TM, 1), lambda t: (t, 0)),
        out_shape=jax.ShapeDtypeStruct((n_tokens, 1), jnp.int32),
        compiler_params=pltpu.CompilerParams(
            dimension_semantics=("arbitrary",)),
        interpret=interpret,
    )(flat_bf, emb_bf, xnrm, enrm)
    return out.reshape(-1)


_CH = 128


def _sc_gather(table, idx):
    info = plsc.get_sparse_core_info()
    nw = info.num_cores * info.num_subcores
    b = idx.shape[0]
    d = table.shape[1]
    bpw = b // nw
    mesh = plsc.VectorSubcoreMesh(core_axis_name="c", subcore_axis_name="s")

    @functools.partial(
        pl.kernel, mesh=mesh,
        out_type=jax.ShapeDtypeStruct((b, d), jnp.float32),
        scratch_types=[
            pltpu.VMEM((_CH,), jnp.int32),
            pltpu.VMEM((_CH, d), jnp.float32),
            pltpu.SemaphoreType.DMA,
        ],
    )
    def gk(table_hbm, idx_hbm, out_hbm, idx_v, rows_v, sem):
        wid = lax.axis_index("s") * info.num_cores + lax.axis_index("c")
        base = wid * bpw

        def body(i, carry):
            off = pl.multiple_of(base + i * _CH, _CH)
            pltpu.sync_copy(idx_hbm.at[pl.ds(off, _CH)], idx_v)
            pltpu.async_copy(table_hbm.at[idx_v], rows_v, sem).wait()
            pltpu.sync_copy(rows_v, out_hbm.at[pl.ds(off, _CH)])
            return carry

        lax.fori_loop(0, bpw // _CH, body, 0)

    return gk(table, idx)


def kernel(inputs, embeddings):
    lat = jnp.transpose(inputs, (0, 2, 3, 1))
    flat = lat.reshape(-1, _D)
    xnrm = jnp.sum(flat ** 2, axis=1, keepdims=True)
    enrm = jnp.sum(embeddings ** 2, axis=1).reshape(1, _NE)
    idx = _tc_argmin(flat.astype(jnp.bfloat16),
                     embeddings.astype(jnp.bfloat16), xnrm, enrm)
    qflat = _sc_gather(embeddings, idx)
    qlat = qflat.reshape(lat.shape)
    quant = lat + lax.stop_gradient(qlat - lat)
    return (jnp.transpose(quant, (0, 3, 1, 2)), lat, qlat)
